# Initial kernel scaffold; baseline (speedup 1.0000x reference)
import functools

import jax
import jax.numpy as jnp
from jax.experimental import pallas as pl
from jax.experimental.pallas import tpu as pltpu

D_MODEL = 1024
D_FF = 1024
E = 64
T = 2048
BT = 128
MAXTILES = T // BT + E
SLOTS = MAXTILES * BT
_CH = 128
_NC = T // _CH
_SCH = 1024
_NSC = SLOTS // _SCH


def _prep_kernel(x_ref, wr_ref, te_ref, rid_ref, q_ref):
    x = x_ref[...]
    wr = wr_ref[...]
    logits = jax.lax.dot_general(
        x, wr, (((1,), (1,)), ((), ())),
        preferred_element_type=jnp.float32,
        precision=jax.lax.Precision.HIGHEST)
    m = jnp.max(logits, axis=1, keepdims=True)
    iota_e = jax.lax.broadcasted_iota(jnp.float32, (T, E), 1)
    e_tok = jnp.min(jnp.where(logits == m, iota_e, float(E)), axis=1,
                    keepdims=True)
    oh = jnp.where(iota_e == e_tok, 1.0, 0.0)
    counts = jnp.sum(oh, axis=0, keepdims=True)
    nt = jnp.floor((counts + (BT - 1)) * (1.0 / BT))
    ue = jnp.where(
        jax.lax.broadcasted_iota(jnp.float32, (E, E), 0)
        <= jax.lax.broadcasted_iota(jnp.float32, (E, E), 1), 1.0, 0.0)
    cumt = jax.lax.dot_general(
        nt, ue, (((1,), (0,)), ((), ())),
        preferred_element_type=jnp.float32)
    po = (cumt - nt) * BT

    lt = jnp.where(
        jax.lax.broadcasted_iota(jnp.float32, (_CH, _CH), 0)
        > jax.lax.broadcasted_iota(jnp.float32, (_CH, _CH), 1), 1.0, 0.0)

    def rank_body(c, base):
        ohc = jax.lax.dynamic_slice(oh, (c * _CH, 0), (_CH, E))
        within = jax.lax.dot_general(
            lt, ohc, (((1,), (0,)), ((), ())),
            preferred_element_type=jnp.float32)
        rank_c = jnp.sum((within + base) * ohc, axis=1)
        po_c = jnp.sum(ohc * po, axis=1)
        q_ref[0, pl.ds(c * _CH, _CH)] = (po_c + rank_c).astype(jnp.int32)
        return base + jnp.sum(ohc, axis=0, keepdims=True)

    jax.lax.fori_loop(0, _NC, rank_body, jnp.zeros((1, E), jnp.float32))

    qv = q_ref[0, :].astype(jnp.float32)
    tval = jax.lax.broadcasted_iota(jnp.float32, (_SCH, T), 1) + 1.0

    def rid_body(c, carry):
        s_iota = (jax.lax.broadcasted_iota(jnp.float32, (_SCH, T), 0)
                  + c.astype(jnp.float32) * _SCH)
        hit = jnp.where(s_iota == qv[None, :], tval, 0.0)
        rid = jnp.sum(hit, axis=1) - 1.0
        rid_ref[0, pl.ds(c * _SCH, _SCH)] = rid.astype(jnp.int32)
        return carry

    jax.lax.fori_loop(0, _NSC, rid_body, 0)

    jt = jax.lax.broadcasted_iota(jnp.float32, (MAXTILES, E), 0)
    te = jnp.sum(jnp.where(cumt <= jt, 1.0, 0.0), axis=1)
    te_ref[0, :] = jnp.minimum(te, float(E - 1)).astype(jnp.int32)


def _gemm_kernel(te_ref, rid_ref, x_ref, w1_ref, v1_ref, w2_ref, out_ref):
    j = pl.program_id(0)
    rows = rid_ref[0, :].astype(jnp.float32)
    tok = jax.lax.broadcasted_iota(jnp.float32, (BT, T), 1)
    g = jnp.where(rows[:, None] == tok, 1.0, 0.0)
    xt = jax.lax.dot_general(
        g, x_ref[...], (((1,), (0,)), ((), ())),
        preferred_element_type=jnp.float32)
    h1 = jax.lax.dot_general(
        xt, w1_ref[0], (((1,), (1,)), ((), ())),
        preferred_element_type=jnp.float32)
    hv = jax.lax.dot_general(
        xt, v1_ref[0], (((1,), (1,)), ((), ())),
        preferred_element_type=jnp.float32)
    h = h1 * jax.lax.logistic(h1) * hv
    o = jax.lax.dot_general(
        h, w2_ref[0], (((1,), (1,)), ((), ())),
        preferred_element_type=jnp.float32)
    contrib = jax.lax.dot_general(
        g, o, (((0,), (0,)), ((), ())),
        preferred_element_type=jnp.float32)

    @pl.when(j == 0)
    def _():
        out_ref[...] = contrib

    @pl.when(j > 0)
    def _():
        out_ref[...] += contrib


def kernel(hidden_states, w_router, w1, v1, w2):
    orig_shape = hidden_states.shape
    x = hidden_states.reshape(T, D_MODEL)

    te, rid, _q = pl.pallas_call(
        _prep_kernel,
        out_shape=(
            jax.ShapeDtypeStruct((1, MAXTILES), jnp.int32),
            jax.ShapeDtypeStruct((1, SLOTS), jnp.int32),
            jax.ShapeDtypeStruct((1, T), jnp.int32),
        ),
    )(x, w_router)

    grid_spec = pltpu.PrefetchScalarGridSpec(
        num_scalar_prefetch=1,
        grid=(MAXTILES,),
        in_specs=[
            pl.BlockSpec((1, BT), lambda j, te_s: (0, j)),
            pl.BlockSpec((T, D_MODEL), lambda j, te_s: (0, 0)),
            pl.BlockSpec((1, D_FF, D_MODEL), lambda j, te_s: (te_s[0, j], 0, 0)),
            pl.BlockSpec((1, D_FF, D_MODEL), lambda j, te_s: (te_s[0, j], 0, 0)),
            pl.BlockSpec((1, D_MODEL, D_FF), lambda j, te_s: (te_s[0, j], 0, 0)),
        ],
        out_specs=pl.BlockSpec((T, D_MODEL), lambda j, te_s: (0, 0)),
    )
    out = pl.pallas_call(
        _gemm_kernel,
        grid_spec=grid_spec,
        out_shape=jax.ShapeDtypeStruct((T, D_MODEL), jnp.float32),
    )(te, rid, x, w1, v1, w2)
    return out.reshape(orig_shape)


# speedup vs baseline: 5.0981x; 5.0981x over previous
import functools

import jax
import jax.numpy as jnp
from jax.experimental import pallas as pl
from jax.experimental.pallas import tpu as pltpu

D_MODEL = 1024
D_FF = 1024
E = 64
T = 2048
BT = 128
MAXTILES = T // BT + E
SLOTS = MAXTILES * BT
_CH = 128
_NC = T // _CH
_SCH = 1024
_NSC = SLOTS // _SCH


def _fiota(shape, dim):
    return jax.lax.broadcasted_iota(jnp.int32, shape, dim).astype(jnp.float32)

def _prep_kernel(x_ref, wr_ref, te_ref, rid_ref, q_ref):
    x = x_ref[...]
    wr = wr_ref[...]
    logits = jax.lax.dot_general(
        x, wr, (((1,), (1,)), ((), ())),
        preferred_element_type=jnp.float32)
    m = jnp.max(logits, axis=1, keepdims=True)
    iota_e = _fiota( (T, E), 1)
    e_tok = jnp.min(jnp.where(logits == m, iota_e, float(E)), axis=1,
                    keepdims=True)
    oh = jnp.where(iota_e == e_tok, 1.0, 0.0)
    counts = jnp.sum(oh, axis=0, keepdims=True)
    nt = jnp.floor((counts + (BT - 1)) * (1.0 / BT))
    ue = jnp.where(
        _fiota( (E, E), 0)
        <= _fiota( (E, E), 1), 1.0, 0.0)
    cumt = jax.lax.dot_general(
        nt, ue, (((1,), (0,)), ((), ())),
        preferred_element_type=jnp.float32)
    po = (cumt - nt) * BT

    lt = jnp.where(
        _fiota( (_CH, _CH), 0)
        > _fiota( (_CH, _CH), 1), 1.0, 0.0)

    base = jnp.zeros((1, E), jnp.float32)
    for c in range(_NC):
        ohc = oh[c * _CH:(c + 1) * _CH, :]
        within = jax.lax.dot_general(
            lt, ohc, (((1,), (0,)), ((), ())),
            preferred_element_type=jnp.float32)
        rank_c = jnp.sum((within + base) * ohc, axis=1)
        po_c = jnp.sum(ohc * po, axis=1)
        q_ref[0, c * _CH:(c + 1) * _CH] = (po_c + rank_c).astype(jnp.int32)
        base = base + jnp.sum(ohc, axis=0, keepdims=True)

    qv = q_ref[0, :].astype(jnp.float32)
    tval = _fiota( (_SCH, T), 1) + 1.0

    def rid_body(c, carry):
        s_iota = (_fiota( (_SCH, T), 0)
                  + c.astype(jnp.float32) * _SCH)
        hit = jnp.where(s_iota == qv[None, :], tval, 0.0)
        rid = jnp.sum(hit, axis=1) - 1.0
        rid_ref[0, pl.ds(c * _SCH, _SCH)] = rid.astype(jnp.int32)
        return carry

    jax.lax.fori_loop(0, _NSC, rid_body, 0)

    jt = _fiota( (MAXTILES, E), 0)
    te = jnp.sum(jnp.where(cumt <= jt, 1.0, 0.0), axis=1)
    te_ref[0, :] = jnp.minimum(te, float(E - 1)).astype(jnp.int32)


def _gemm_kernel(te_ref, rid_ref, x_ref, w1_ref, v1_ref, w2_ref, out_ref):
    j = pl.program_id(0)
    rows = rid_ref[0, :].astype(jnp.float32)
    tok = _fiota( (BT, T), 1)
    g = jnp.where(rows[:, None] == tok, 1.0, 0.0)
    xt = jax.lax.dot_general(
        g, x_ref[...], (((1,), (0,)), ((), ())),
        preferred_element_type=jnp.float32)
    h1 = jax.lax.dot_general(
        xt, w1_ref[0], (((1,), (1,)), ((), ())),
        preferred_element_type=jnp.float32)
    hv = jax.lax.dot_general(
        xt, v1_ref[0], (((1,), (1,)), ((), ())),
        preferred_element_type=jnp.float32)
    h = h1 * jax.lax.logistic(h1) * hv
    o = jax.lax.dot_general(
        h, w2_ref[0], (((1,), (1,)), ((), ())),
        preferred_element_type=jnp.float32)
    contrib = jax.lax.dot_general(
        g, o, (((0,), (0,)), ((), ())),
        preferred_element_type=jnp.float32)

    @pl.when(j == 0)
    def _():
        out_ref[...] = contrib

    @pl.when(j > 0)
    def _():
        out_ref[...] += contrib


def kernel(hidden_states, w_router, w1, v1, w2):
    orig_shape = hidden_states.shape
    x = hidden_states.reshape(T, D_MODEL)

    te, rid, _q = pl.pallas_call(
        _prep_kernel,
        out_shape=(
            jax.ShapeDtypeStruct((1, MAXTILES), jnp.int32),
            jax.ShapeDtypeStruct((1, SLOTS), jnp.int32),
            jax.ShapeDtypeStruct((1, T), jnp.int32),
        ),
    )(x, w_router)

    grid_spec = pltpu.PrefetchScalarGridSpec(
        num_scalar_prefetch=1,
        grid=(MAXTILES,),
        in_specs=[
            pl.BlockSpec((1, BT), lambda j, te_s: (0, j)),
            pl.BlockSpec((T, D_MODEL), lambda j, te_s: (0, 0)),
            pl.BlockSpec((1, D_FF, D_MODEL), lambda j, te_s: (te_s[0, j], 0, 0)),
            pl.BlockSpec((1, D_FF, D_MODEL), lambda j, te_s: (te_s[0, j], 0, 0)),
            pl.BlockSpec((1, D_MODEL, D_FF), lambda j, te_s: (te_s[0, j], 0, 0)),
        ],
        out_specs=pl.BlockSpec((T, D_MODEL), lambda j, te_s: (0, 0)),
    )
    out = pl.pallas_call(
        _gemm_kernel,
        grid_spec=grid_spec,
        out_shape=jax.ShapeDtypeStruct((T, D_MODEL), jnp.float32),
    )(te, rid, x, w1, v1, w2)
    return out.reshape(orig_shape)
